# baseline (speedup 1.0000x reference)
import jax
import jax.numpy as jnp
from jax.experimental import pallas as pl
from jax.experimental.pallas import tpu as pltpu

B, N, C = 8, 20000, 80
HC = C // 2
NCHUNK = 2 * B
NBUF = 3


def _manual_kernel(lt_ref, ct_ref, rt_ref, pt_ref, lo_ref, bo_ref,
                   lbuf, obuf, cbuf, rbuf, pbuf, bbuf,
                   sin, sout, saux, sbox):
    cp_c = pltpu.make_async_copy(ct_ref, cbuf, saux.at[0])
    cp_r = pltpu.make_async_copy(rt_ref, rbuf, saux.at[1])
    cp_p = pltpu.make_async_copy(pt_ref, pbuf, saux.at[2])
    cp_c.start()
    cp_r.start()
    cp_p.start()

    def in_copy(i, slot):
        return pltpu.make_async_copy(
            lt_ref.at[i // 2, pl.ds((i % 2) * HC, HC)], lbuf.at[slot],
            sin.at[slot])

    def out_copy(i, slot):
        return pltpu.make_async_copy(
            obuf.at[slot], lo_ref.at[i // 2, pl.ds((i % 2) * HC, HC)],
            sout.at[slot])

    for i in range(NBUF):
        in_copy(i, i).start()

    cp_r.wait()
    cp_p.wait()
    r = rbuf[...]
    px = pbuf[0:1, :][None]
    py = pbuf[1:2, :][None]
    row = jax.lax.broadcasted_iota(jnp.int32, r.shape, 1)
    sign = jnp.where(row >= 2, 1.0, -1.0).astype(jnp.float32)
    pts4 = jnp.where(row % 2 == 0, px, py)
    bbuf[...] = jnp.clip(pts4 + sign * r, 0.0, 1.0)
    box_dma = pltpu.make_async_copy(bbuf, bo_ref, sbox)
    box_dma.start()
    cp_c.wait()

    for i in range(NCHUNK):
        slot = i % NBUF
        in_copy(i, slot).wait()
        if i >= NBUF:
            out_copy(i - NBUF, slot).wait()
        c = cbuf[i // 2]
        obuf[slot] = jnp.sqrt(lbuf[slot] * c[None, :])
        out_copy(i, slot).start()
        if i + NBUF < NCHUNK:
            in_copy(i + NBUF, slot).start()

    for i in range(NCHUNK - NBUF, NCHUNK):
        out_copy(i, i % NBUF).wait()
    box_dma.wait()


def kernel(logits, regress, points, centerness):
    lt = jnp.transpose(logits, (0, 2, 1))
    rt = jnp.transpose(regress, (0, 2, 1))
    pt = jnp.transpose(points, (1, 0))
    ct = jnp.transpose(centerness, (0, 2, 1))
    ct2 = ct.reshape(B, N)

    hbm = pl.BlockSpec(memory_space=pltpu.HBM)
    out = pl.pallas_call(
        _manual_kernel,
        in_specs=[hbm, hbm, hbm, hbm],
        out_specs=[hbm, hbm],
        out_shape=[
            jax.ShapeDtypeStruct((B, C, N), jnp.float32),
            jax.ShapeDtypeStruct((B, 4, N), jnp.float32),
        ],
        scratch_shapes=[
            pltpu.VMEM((NBUF, HC, N), jnp.float32),
            pltpu.VMEM((NBUF, HC, N), jnp.float32),
            pltpu.VMEM((B, N), jnp.float32),
            pltpu.VMEM((B, 4, N), jnp.float32),
            pltpu.VMEM((2, N), jnp.float32),
            pltpu.VMEM((B, 4, N), jnp.float32),
            pltpu.SemaphoreType.DMA((NBUF,)),
            pltpu.SemaphoreType.DMA((NBUF,)),
            pltpu.SemaphoreType.DMA((3,)),
            pltpu.SemaphoreType.DMA,
        ],
    )(lt, ct2, rt, pt)
    return (jnp.transpose(out[0], (0, 2, 1)), jnp.transpose(out[1], (0, 2, 1)))


# speedup vs baseline: 1.0036x; 1.0036x over previous
import jax
import jax.numpy as jnp
from jax.experimental import pallas as pl
from jax.experimental.pallas import tpu as pltpu

B, N, C = 8, 20000, 80
CPB = 5
HC = C // CPB
NCHUNK = CPB * B
NBUF = 6


def _manual_kernel(lt_ref, ct_ref, rt_ref, pt_ref, lo_ref, bo_ref,
                   lbuf, obuf, cbuf, rbuf, pbuf, bbuf,
                   sin, sout, saux, sbox):
    cp_c = pltpu.make_async_copy(ct_ref, cbuf, saux.at[0])
    cp_r = pltpu.make_async_copy(rt_ref, rbuf, saux.at[1])
    cp_p = pltpu.make_async_copy(pt_ref, pbuf, saux.at[2])
    cp_c.start()
    cp_r.start()
    cp_p.start()

    def in_copy(i, slot):
        return pltpu.make_async_copy(
            lt_ref.at[i // CPB, pl.ds((i % CPB) * HC, HC)], lbuf.at[slot],
            sin.at[slot])

    def out_copy(i, slot):
        return pltpu.make_async_copy(
            obuf.at[slot], lo_ref.at[i // CPB, pl.ds((i % CPB) * HC, HC)],
            sout.at[slot])

    for i in range(NBUF):
        in_copy(i, i).start()

    cp_r.wait()
    cp_p.wait()
    r = rbuf[...]
    px = pbuf[0:1, :][None]
    py = pbuf[1:2, :][None]
    row = jax.lax.broadcasted_iota(jnp.int32, r.shape, 1)
    sign = jnp.where(row >= 2, 1.0, -1.0).astype(jnp.float32)
    pts4 = jnp.where(row % 2 == 0, px, py)
    bbuf[...] = jnp.clip(pts4 + sign * r, 0.0, 1.0)
    box_dma = pltpu.make_async_copy(bbuf, bo_ref, sbox)
    box_dma.start()
    cp_c.wait()

    for i in range(NCHUNK):
        slot = i % NBUF
        in_copy(i, slot).wait()
        if i >= NBUF:
            out_copy(i - NBUF, slot).wait()
        c = cbuf[i // CPB]
        obuf[slot] = jnp.sqrt(lbuf[slot] * c[None, :])
        out_copy(i, slot).start()
        if i + NBUF < NCHUNK:
            in_copy(i + NBUF, slot).start()

    for i in range(NCHUNK - NBUF, NCHUNK):
        out_copy(i, i % NBUF).wait()
    box_dma.wait()


def kernel(logits, regress, points, centerness):
    lt = jnp.transpose(logits, (0, 2, 1))
    rt = jnp.transpose(regress, (0, 2, 1))
    pt = jnp.transpose(points, (1, 0))
    ct = jnp.transpose(centerness, (0, 2, 1))
    ct2 = ct.reshape(B, N)

    hbm = pl.BlockSpec(memory_space=pltpu.HBM)
    out = pl.pallas_call(
        _manual_kernel,
        in_specs=[hbm, hbm, hbm, hbm],
        out_specs=[hbm, hbm],
        out_shape=[
            jax.ShapeDtypeStruct((B, C, N), jnp.float32),
            jax.ShapeDtypeStruct((B, 4, N), jnp.float32),
        ],
        scratch_shapes=[
            pltpu.VMEM((NBUF, HC, N), jnp.float32),
            pltpu.VMEM((NBUF, HC, N), jnp.float32),
            pltpu.VMEM((B, N), jnp.float32),
            pltpu.VMEM((B, 4, N), jnp.float32),
            pltpu.VMEM((2, N), jnp.float32),
            pltpu.VMEM((B, 4, N), jnp.float32),
            pltpu.SemaphoreType.DMA((NBUF,)),
            pltpu.SemaphoreType.DMA((NBUF,)),
            pltpu.SemaphoreType.DMA((3,)),
            pltpu.SemaphoreType.DMA,
        ],
    )(lt, ct2, rt, pt)
    return (jnp.transpose(out[0], (0, 2, 1)), jnp.transpose(out[1], (0, 2, 1)))


# speedup vs baseline: 1.0349x; 1.0312x over previous
import jax
import jax.numpy as jnp
from jax.experimental import pallas as pl
from jax.experimental.pallas import tpu as pltpu

B, N, C = 8, 20000, 80
CSPLIT = 1
BC = C // CSPLIT


def _fused_kernel(logits_ref, cent_ref, regress_ref, pts_ref,
                  logits_out_ref, boxes_out_ref):
    l = logits_ref[...]
    c = cent_ref[...]
    logits_out_ref[...] = jnp.sqrt(l * c)

    @pl.when(pl.program_id(1) == 0)
    def _():
        r = regress_ref[...]
        px = pts_ref[0:1, :][None]
        py = pts_ref[1:2, :][None]
        row = jax.lax.broadcasted_iota(jnp.int32, r.shape, 1)
        sign = jnp.where(row >= 2, 1.0, -1.0).astype(jnp.float32)
        pts4 = jnp.where(row % 2 == 0, px, py)
        boxes_out_ref[...] = jnp.clip(pts4 + sign * r, 0.0, 1.0)


def kernel(logits, regress, points, centerness):
    lt = jnp.transpose(logits, (0, 2, 1))
    rt = jnp.transpose(regress, (0, 2, 1))
    pt = jnp.transpose(points, (1, 0))
    ct = jnp.transpose(centerness, (0, 2, 1))

    out = pl.pallas_call(
        _fused_kernel,
        grid=(B, CSPLIT),
        in_specs=[
            pl.BlockSpec((1, BC, N), lambda b, j: (b, j, 0)),
            pl.BlockSpec((1, 1, N), lambda b, j: (b, 0, 0)),
            pl.BlockSpec((1, 4, N), lambda b, j: (b, 0, 0)),
            pl.BlockSpec((2, N), lambda b, j: (0, 0)),
        ],
        out_specs=[
            pl.BlockSpec((1, BC, N), lambda b, j: (b, j, 0)),
            pl.BlockSpec((1, 4, N), lambda b, j: (b, 0, 0)),
        ],
        out_shape=[
            jax.ShapeDtypeStruct((B, C, N), jnp.float32),
            jax.ShapeDtypeStruct((B, 4, N), jnp.float32),
        ],
        compiler_params=pltpu.CompilerParams(
            dimension_semantics=("parallel", "arbitrary"),
        ),
    )(lt, ct, rt, pt)
    return (jnp.transpose(out[0], (0, 2, 1)), jnp.transpose(out[1], (0, 2, 1)))
